# baseline (speedup 1.0000x reference)
import functools

import jax
import jax.numpy as jnp
from jax import lax
from jax.experimental import pallas as pl
from jax.experimental.pallas import tpu as pltpu
from jax.experimental.pallas import tpu_sc as plsc

NUM_SEGMENTS = 200
ACC_PAD = 208
COMP = 9
NC = 2
NS = 16
NXT = 8
SROWS = 8
XW = 128
UNROLL = 8


def _sc_segment_sum(ids3, srcp, bz, h, w):
    bpc = bz // NC
    yrows = h // 2
    nst = yrows // SROWS
    nxv = XW // 16
    assert w == NXT * XW and yrows % (2 * SROWS) == 0

    mesh = plsc.VectorSubcoreMesh(core_axis_name="c", subcore_axis_name="s")

    @functools.partial(
        pl.kernel,
        out_type=jax.ShapeDtypeStruct((NC, NS, bpc * COMP, ACC_PAD),
                                      jnp.float32),
        mesh=mesh,
        compiler_params=pltpu.CompilerParams(
            use_tc_tiling_on_sc=True, needs_layout_passes=False),
        scratch_types=[
            [[pltpu.VMEM((ACC_PAD,), jnp.float32) for _ in range(COMP)]
             for _ in range(bpc)],
            pltpu.VMEM((bpc * COMP, ACC_PAD), jnp.float32),
            [[pltpu.VMEM((SROWS, XW), jnp.float32) for _ in range(COMP)]
             for _ in range(2)],
            pltpu.VMEM((yrows, XW), jnp.int32),
            [pltpu.SemaphoreType.DMA for _ in range(2)],
            pltpu.SemaphoreType.DMA,
        ],
    )
    def seg_sum(ids_hbm, src_hbm, out_hbm, accs, stage, pbufs, ibuf,
                psems, isem):
        c = lax.axis_index("c")
        s = lax.axis_index("s")
        y0 = pl.multiple_of((s // NXT) * yrows, SROWS)
        x0 = pl.multiple_of((s % NXT) * XW, XW)
        zero16 = jnp.zeros((16,), jnp.float32)

        def zero_body(i, _):
            for lb in range(bpc):
                for k in range(COMP):
                    accs[lb][k][pl.ds(i * 16, 16)] = zero16
            return 0

        lax.fori_loop(0, ACC_PAD // 16, zero_body, 0)

        def start_ids(lb):
            return pltpu.async_copy(
                ids_hbm.at[c * bpc + lb, pl.ds(y0, yrows), pl.ds(x0, XW)],
                ibuf, isem)

        def start_stripe(lb, st, j):
            for k in range(COMP):
                pltpu.async_copy(
                    src_hbm.at[c * bpc + lb, k,
                               pl.ds(pl.multiple_of(y0 + st * SROWS, SROWS),
                                     SROWS),
                               pl.ds(x0, XW)],
                    pbufs[j][k], psems[j])

        def wait_stripe(j):
            for k in range(COMP):
                pltpu.make_async_copy(
                    src_hbm.at[0, 0, pl.ds(0, SROWS), pl.ds(0, XW)],
                    pbufs[j][k], psems[j]).wait()

        id_dma = start_ids(0)
        start_stripe(0, 0, 0)
        id_dma.wait()

        for lb in range(bpc):

            def stripe_pair(tt, _, lb=lb):
                for j in range(2):
                    st = tt * 2 + j
                    nxt_st = st + 1

                    @pl.when(nxt_st < nst)
                    def _():
                        start_stripe(lb, nxt_st, 1 - j)

                    wait_stripe(j)

                    for yy in range(SROWS):

                        @plsc.parallel_loop(0, nxv, unroll=UNROLL)
                        def _(xv, j=j, lb=lb, yy=yy, st=st):
                            ids16 = ibuf[st * SROWS + yy,
                                         pl.ds(xv * 16, 16)]
                            for k in range(COMP):
                                vals = pbufs[j][k][yy, pl.ds(xv * 16, 16)]
                                plsc.addupdate_scatter(accs[lb][k], [ids16],
                                                       vals)
                return 0

            lax.fori_loop(0, nst // 2, stripe_pair, 0)

            if lb + 1 < bpc:
                nid = start_ids(lb + 1)
                start_stripe(lb + 1, 0, 0)
                nid.wait()

        def stage_body(i, _):
            for lb in range(bpc):
                for k in range(COMP):
                    stage[lb * COMP + k, pl.ds(i * 16, 16)] = (
                        accs[lb][k][pl.ds(i * 16, 16)])
            return 0

        lax.fori_loop(0, ACC_PAD // 16, stage_body, 0)
        pltpu.sync_copy(stage, out_hbm.at[c, s])

    return seg_sum(ids3, srcp)


def _tc_finish(partials):
    nc, ns, bk = partials.shape[:3]
    bpc = bk // COMP

    def body(x_ref, o_ref):
        summed = jnp.sum(x_ref[...], axis=1)
        trimmed = summed[:, :, :NUM_SEGMENTS] * jnp.float32(1.1)
        split = trimmed.reshape(nc, bpc, COMP, NUM_SEGMENTS)
        swapped = jnp.swapaxes(split, 2, 3)
        o_ref[...] = swapped.reshape(nc * bpc, NUM_SEGMENTS, COMP)

    return pl.pallas_call(
        body,
        out_shape=jax.ShapeDtypeStruct((nc * bpc, NUM_SEGMENTS, COMP),
                                       jnp.float32),
    )(partials)


def kernel(instance, compsrc, maxinsnum):
    bz, _, h, w = instance.shape
    ids3 = instance.reshape(bz, h, w)
    srcp = compsrc.transpose(0, 3, 4, 1, 2).reshape(bz, COMP, h, w)
    partials = _sc_segment_sum(ids3, srcp, bz, h, w)
    out = _tc_finish(partials)
    return out.reshape(bz, NUM_SEGMENTS, 3, 3)


# speedup vs baseline: 1.1395x; 1.1395x over previous
import functools

import jax
import jax.numpy as jnp
from jax import lax
from jax.experimental import pallas as pl
from jax.experimental.pallas import tpu as pltpu
from jax.experimental.pallas import tpu_sc as plsc

NUM_SEGMENTS = 200
ACC_PAD = 208
COMP = 9
NC = 2
NS = 16
NXT = 8
SROWS = 8
XW = 128
UNROLL = 2


def _sc_segment_sum(ids3, srcp, bz, h, w):
    bpc = bz // NC
    yrows = h // 2
    nst = yrows // SROWS
    nxv = XW // 16
    assert w == NXT * XW and yrows % (2 * SROWS) == 0

    mesh = plsc.VectorSubcoreMesh(core_axis_name="c", subcore_axis_name="s")

    @functools.partial(
        pl.kernel,
        out_type=jax.ShapeDtypeStruct((NC, NS, bpc * COMP, ACC_PAD),
                                      jnp.float32),
        mesh=mesh,
        compiler_params=pltpu.CompilerParams(
            use_tc_tiling_on_sc=True, needs_layout_passes=False),
        scratch_types=[
            [[pltpu.VMEM((ACC_PAD,), jnp.float32) for _ in range(COMP)]
             for _ in range(bpc)],
            pltpu.VMEM((bpc * COMP, ACC_PAD), jnp.float32),
            [[pltpu.VMEM((SROWS, XW), jnp.float32) for _ in range(COMP)]
             for _ in range(2)],
            pltpu.VMEM((yrows, XW), jnp.int32),
            [pltpu.SemaphoreType.DMA for _ in range(2)],
            pltpu.SemaphoreType.DMA,
        ],
    )
    def seg_sum(ids_hbm, src_hbm, out_hbm, accs, stage, pbufs, ibuf,
                psems, isem):
        c = lax.axis_index("c")
        s = lax.axis_index("s")
        y0 = pl.multiple_of((s // NXT) * yrows, SROWS)
        x0 = pl.multiple_of((s % NXT) * XW, XW)
        zero16 = jnp.zeros((16,), jnp.float32)

        def zero_body(i, _):
            for lb in range(bpc):
                for k in range(COMP):
                    accs[lb][k][pl.ds(i * 16, 16)] = zero16
            return 0

        lax.fori_loop(0, ACC_PAD // 16, zero_body, 0)

        def start_ids(lb):
            return pltpu.async_copy(
                ids_hbm.at[c * bpc + lb, pl.ds(y0, yrows), pl.ds(x0, XW)],
                ibuf, isem)

        def start_stripe(lb, st, j):
            for k in range(COMP):
                pltpu.async_copy(
                    src_hbm.at[c * bpc + lb, k,
                               pl.ds(pl.multiple_of(y0 + st * SROWS, SROWS),
                                     SROWS),
                               pl.ds(x0, XW)],
                    pbufs[j][k], psems[j])

        def wait_stripe(j):
            for k in range(COMP):
                pltpu.make_async_copy(
                    src_hbm.at[0, 0, pl.ds(0, SROWS), pl.ds(0, XW)],
                    pbufs[j][k], psems[j]).wait()

        id_dma = start_ids(0)
        start_stripe(0, 0, 0)
        id_dma.wait()

        for lb in range(bpc):

            def stripe_pair(tt, _, lb=lb):
                for j in range(2):
                    st = tt * 2 + j
                    nxt_st = st + 1

                    @pl.when(nxt_st < nst)
                    def _():
                        start_stripe(lb, nxt_st, 1 - j)

                    wait_stripe(j)

                    for yy in range(SROWS):

                        @plsc.parallel_loop(0, nxv, unroll=UNROLL)
                        def _(xv, j=j, lb=lb, yy=yy, st=st):
                            ids16 = ibuf[st * SROWS + yy,
                                         pl.ds(xv * 16, 16)]
                            for k in range(COMP):
                                vals = pbufs[j][k][yy, pl.ds(xv * 16, 16)]
                                plsc.addupdate_scatter(accs[lb][k], [ids16],
                                                       vals)
                return 0

            lax.fori_loop(0, nst // 2, stripe_pair, 0)

            if lb + 1 < bpc:
                nid = start_ids(lb + 1)
                start_stripe(lb + 1, 0, 0)
                nid.wait()

        def stage_body(i, _):
            for lb in range(bpc):
                for k in range(COMP):
                    stage[lb * COMP + k, pl.ds(i * 16, 16)] = (
                        accs[lb][k][pl.ds(i * 16, 16)])
            return 0

        lax.fori_loop(0, ACC_PAD // 16, stage_body, 0)
        pltpu.sync_copy(stage, out_hbm.at[c, s])

    return seg_sum(ids3, srcp)


def _tc_finish(partials):
    nc, ns, bk = partials.shape[:3]
    bpc = bk // COMP

    def body(x_ref, o_ref):
        summed = jnp.sum(x_ref[...], axis=1)
        trimmed = summed[:, :, :NUM_SEGMENTS] * jnp.float32(1.1)
        split = trimmed.reshape(nc, bpc, COMP, NUM_SEGMENTS)
        swapped = jnp.swapaxes(split, 2, 3)
        o_ref[...] = swapped.reshape(nc * bpc, NUM_SEGMENTS, COMP)

    return pl.pallas_call(
        body,
        out_shape=jax.ShapeDtypeStruct((nc * bpc, NUM_SEGMENTS, COMP),
                                       jnp.float32),
    )(partials)


def kernel(instance, compsrc, maxinsnum):
    bz, _, h, w = instance.shape
    ids3 = instance.reshape(bz, h, w)
    srcp = compsrc.transpose(0, 3, 4, 1, 2).reshape(bz, COMP, h, w)
    partials = _sc_segment_sum(ids3, srcp, bz, h, w)
    out = _tc_finish(partials)
    return out.reshape(bz, NUM_SEGMENTS, 3, 3)


# speedup vs baseline: 1.6924x; 1.4853x over previous
import functools

import jax
import jax.numpy as jnp
from jax import lax
from jax.experimental import pallas as pl
from jax.experimental.pallas import tpu as pltpu
from jax.experimental.pallas import tpu_sc as plsc

NUM_SEGMENTS = 200
ACC_PAD = 208
COMP = 9
NC = 2
NS = 16
NXT = 8
SROWS = 8
XW = 128
UNROLL = 4


def _sc_segment_sum(ids3, srcp, bz, h, w):
    bpc = bz // NC
    yrows = h // 2
    nst = yrows // SROWS
    nxv = XW // 16
    assert w == NXT * XW and yrows % (2 * SROWS) == 0

    mesh = plsc.VectorSubcoreMesh(core_axis_name="c", subcore_axis_name="s")

    @functools.partial(
        pl.kernel,
        out_type=jax.ShapeDtypeStruct((NC, NS, bpc * COMP, ACC_PAD),
                                      jnp.float32),
        mesh=mesh,
        compiler_params=pltpu.CompilerParams(
            use_tc_tiling_on_sc=True, needs_layout_passes=False),
        scratch_types=[
            [[pltpu.VMEM((ACC_PAD,), jnp.float32) for _ in range(COMP)]
             for _ in range(bpc)],
            pltpu.VMEM((bpc * COMP, ACC_PAD), jnp.float32),
            [[pltpu.VMEM((SROWS, XW), jnp.float32) for _ in range(COMP)]
             for _ in range(2)],
            pltpu.VMEM((yrows, XW), jnp.int32),
            [pltpu.SemaphoreType.DMA for _ in range(2)],
            pltpu.SemaphoreType.DMA,
        ],
    )
    def seg_sum(ids_hbm, src_hbm, out_hbm, accs, stage, pbufs, ibuf,
                psems, isem):
        c = lax.axis_index("c")
        s = lax.axis_index("s")
        y0 = pl.multiple_of((s // NXT) * yrows, SROWS)
        x0 = pl.multiple_of((s % NXT) * XW, XW)
        zero16 = jnp.zeros((16,), jnp.float32)

        def zero_body(i, _):
            for lb in range(bpc):
                for k in range(COMP):
                    accs[lb][k][pl.ds(i * 16, 16)] = zero16
            return 0

        lax.fori_loop(0, ACC_PAD // 16, zero_body, 0)

        def start_ids(lb):
            return pltpu.async_copy(
                ids_hbm.at[c * bpc + lb, pl.ds(y0, yrows), pl.ds(x0, XW)],
                ibuf, isem)

        def start_stripe(lb, st, j):
            for k in range(COMP):
                pltpu.async_copy(
                    src_hbm.at[c * bpc + lb, k,
                               pl.ds(pl.multiple_of(y0 + st * SROWS, SROWS),
                                     SROWS),
                               pl.ds(x0, XW)],
                    pbufs[j][k], psems[j])

        def wait_stripe(j):
            for k in range(COMP):
                pltpu.make_async_copy(
                    src_hbm.at[0, 0, pl.ds(0, SROWS), pl.ds(0, XW)],
                    pbufs[j][k], psems[j]).wait()

        id_dma = start_ids(0)
        start_stripe(0, 0, 0)
        id_dma.wait()

        for lb in range(bpc):

            def stripe_pair(tt, _, lb=lb):
                for j in range(2):
                    st = tt * 2 + j
                    nxt_st = st + 1

                    @pl.when(nxt_st < nst)
                    def _():
                        start_stripe(lb, nxt_st, 1 - j)

                    wait_stripe(j)

                    @plsc.parallel_loop(0, SROWS * nxv, unroll=UNROLL)
                    def _(v, j=j, lb=lb, st=st):
                        yy = v // nxv
                        x16 = (v % nxv) * 16
                        ids16 = ibuf[st * SROWS + yy, pl.ds(x16, 16)]
                        for k in range(COMP):
                            vals = pbufs[j][k][yy, pl.ds(x16, 16)]
                            plsc.addupdate_scatter(accs[lb][k], [ids16],
                                                   vals)
                return 0

            lax.fori_loop(0, nst // 2, stripe_pair, 0)

            if lb + 1 < bpc:
                nid = start_ids(lb + 1)
                start_stripe(lb + 1, 0, 0)
                nid.wait()

        def stage_body(i, _):
            for lb in range(bpc):
                for k in range(COMP):
                    stage[lb * COMP + k, pl.ds(i * 16, 16)] = (
                        accs[lb][k][pl.ds(i * 16, 16)])
            return 0

        lax.fori_loop(0, ACC_PAD // 16, stage_body, 0)
        pltpu.sync_copy(stage, out_hbm.at[c, s])

    return seg_sum(ids3, srcp)


def _tc_finish(partials):
    nc, ns, bk = partials.shape[:3]
    bpc = bk // COMP

    def body(x_ref, o_ref):
        summed = jnp.sum(x_ref[...], axis=1)
        trimmed = summed[:, :, :NUM_SEGMENTS] * jnp.float32(1.1)
        split = trimmed.reshape(nc, bpc, COMP, NUM_SEGMENTS)
        swapped = jnp.swapaxes(split, 2, 3)
        o_ref[...] = swapped.reshape(nc * bpc, NUM_SEGMENTS, COMP)

    return pl.pallas_call(
        body,
        out_shape=jax.ShapeDtypeStruct((nc * bpc, NUM_SEGMENTS, COMP),
                                       jnp.float32),
    )(partials)


def kernel(instance, compsrc, maxinsnum):
    bz, _, h, w = instance.shape
    ids3 = instance.reshape(bz, h, w)
    srcp = compsrc.transpose(0, 3, 4, 1, 2).reshape(bz, COMP, h, w)
    partials = _sc_segment_sum(ids3, srcp, bz, h, w)
    out = _tc_finish(partials)
    return out.reshape(bz, NUM_SEGMENTS, 3, 3)
